# baseline (speedup 1.0000x reference)
import functools

import jax
import jax.numpy as jnp
from jax import lax
from jax.experimental import pallas as pl
from jax.experimental.pallas import tpu as pltpu
from jax.experimental.pallas import tpu_sc as plsc

NUM_CLASSES = 1024
EMB = 512
N_SUPPORT = 32768
N_QUERY = 16384

NC = 2
NS = 16
L = 16
NW = NC * NS
TC_ROWS = N_SUPPORT // 2
SC_ROWS = N_SUPPORT - TC_ROWS
ROWS_PER_TILE = SC_ROWS // NW
CHUNKS = EMB // L
GB = 32
NBATCH = ROWS_PER_TILE // GB
NBUF = 4
CLS_STRIPE = NUM_CLASSES // NS
CW = 16


def _sc_segment_body(feat_hbm, lbl_hbm, sums_hbm, counts_hbm,
                     in0, in1, out0, out1,
                     lbi0, lbi1, lbo0, lbo1, ones_v,
                     sums_sh, counts_sh, sg0, sg1, ss0, ss1):
    c = lax.axis_index("c")
    s = lax.axis_index("s")
    tid = c * NS + s
    row_base = TC_ROWS + tid * ROWS_PER_TILE
    acc_base = c * NUM_CLASSES

    ins = [in0, in1]
    outs = [out0, out1]
    lbis = [lbi0, lbi1]
    lbos = [lbo0, lbo1]
    sgs = [sg0, sg1]
    sss = [ss0, ss1]

    zeros = jnp.zeros((L,), jnp.float32)

    def zero_out0(r, _):
        for k in range(CHUNKS):
            out0[r, pl.ds(k * L, L)] = zeros
        for k in range(CW // L):
            ones_v[r, pl.ds(k * L, L)] = zeros
        return 0
    lax.fori_loop(0, GB, zero_out0, 0)
    stripe = s * CLS_STRIPE
    pltpu.sync_copy(out0, sums_sh.at[pl.ds(stripe, GB)])
    pltpu.sync_copy(out0, sums_sh.at[pl.ds(stripe + GB, GB)])
    pltpu.sync_copy(ones_v, counts_sh.at[pl.ds(stripe, GB)])
    pltpu.sync_copy(ones_v, counts_sh.at[pl.ds(stripe + GB, GB)])

    def fill_ones(r, _):
        for k in range(CW // L):
            ones_v[r, pl.ds(k * L, L)] = jnp.full((L,), 1.0, jnp.float32)
        return 0
    lax.fori_loop(0, GB, fill_ones, 0)

    plsc.subcore_barrier()

    def issue_gather(b, B):
        blk = (row_base + b * GB) // 8
        pltpu.async_copy(feat_hbm.at[pl.ds(blk, GB // 8)], ins[B], sgs[B])
        pltpu.async_copy(lbl_hbm.at[pl.ds(row_base + b * GB, GB)],
                         lbis[B], sgs[B])

    def wait_gather(b, B):
        blk = (row_base + b * GB) // 8
        pltpu.make_async_copy(feat_hbm.at[pl.ds(blk, GB // 8)],
                              ins[B], sgs[B]).wait()
        pltpu.make_async_copy(lbl_hbm.at[pl.ds(row_base + b * GB, GB)],
                              lbis[B], sgs[B]).wait()

    def issue_scatter(B):
        pltpu.async_copy(outs[B], sums_sh.at[lbos[B]], sss[B], add=True)
        pltpu.async_copy(ones_v, counts_sh.at[lbos[B]], sss[B], add=True)

    def wait_scatter(B):
        pltpu.make_async_copy(outs[B], sums_sh.at[lbos[B]], sss[B]).wait()
        pltpu.make_async_copy(ones_v, counts_sh.at[lbos[B]], sss[B]).wait()

    def normalize(BI, BO):
        for g in range(GB // L):
            lbos[BO][pl.ds(g * L, L)] = lbis[BI][pl.ds(g * L, L)]

        def block_body(rb, _):
            for rs in range(8):
                r = rb * 8 + rs
                vs = []
                parts = []
                for k in range(CHUNKS):
                    v = ins[BI][rb, k // 8, rs, pl.ds((k % 8) * L, L)]
                    vs.append(v)
                    parts.append(v * v)
                while len(parts) > 1:
                    rest = [parts[-1]] if len(parts) % 2 else []
                    parts = [parts[i2] + parts[i2 + 1]
                             for i2 in range(0, len(parts) - 1, 2)] + rest
                tv = jnp.broadcast_to(jnp.sum(parts[0]), (L,))
                iv = plsc.bitcast(tv, jnp.int32)
                y = plsc.bitcast(jnp.int32(0x5F3759DF) - (iv >> 1),
                                 jnp.float32)
                for _ in range(3):
                    y = y * (jnp.float32(1.5) - jnp.float32(0.5) * tv * y * y)
                for k in range(CHUNKS):
                    outs[BO][r, pl.ds(k * L, L)] = vs[k] * y
            return 0

        lax.fori_loop(0, GB // 8, block_body, 0)

    issue_gather(0, 0)
    issue_gather(1, 1)

    def pair_body(p, _):
        for sec in range(2):
            b = 2 * p + sec
            wait_gather(b, sec)

            @pl.when(b >= 2)
            def _(sec=sec):
                wait_scatter(sec)
            normalize(sec, sec)
            issue_scatter(sec)

            @pl.when(b + 2 < NBATCH)
            def _(b=b, sec=sec):
                issue_gather(b + 2, sec)
        return 0

    lax.fori_loop(0, NBATCH // 2, pair_body, 0)
    wait_scatter(0)
    wait_scatter(1)

    plsc.subcore_barrier()
    pltpu.sync_copy(sums_sh.at[pl.ds(stripe, CLS_STRIPE)],
                    sums_hbm.at[pl.ds(acc_base + stripe, CLS_STRIPE)])
    pltpu.sync_copy(counts_sh.at[pl.ds(stripe, CLS_STRIPE)],
                    counts_hbm.at[pl.ds(acc_base + stripe, CLS_STRIPE)])


def _sc_segment(feat_t, lbl):
    mesh = plsc.VectorSubcoreMesh(core_axis_name="c", subcore_axis_name="s",
                                  num_cores=NC, num_subcores=NS)
    return pl.kernel(
        _sc_segment_body,
        out_type=(jax.ShapeDtypeStruct((NC * NUM_CLASSES, EMB), jnp.float32),
                  jax.ShapeDtypeStruct((NC * NUM_CLASSES, CW), jnp.float32)),
        mesh=mesh,
        compiler_params=pltpu.CompilerParams(needs_layout_passes=False,
                                             use_tc_tiling_on_sc=False),
        scratch_types=(
            [pltpu.VMEM((GB // 8, EMB // 128, 8, 128), jnp.float32)
             for _ in range(2)]
            + [pltpu.VMEM((GB, EMB), jnp.float32) for _ in range(2)]
            + [pltpu.VMEM((GB,), jnp.int32) for _ in range(4)]
            + [pltpu.VMEM((GB, CW), jnp.float32)]
            + [pltpu.VMEM_SHARED((NUM_CLASSES, EMB), jnp.float32),
               pltpu.VMEM_SHARED((NUM_CLASSES, CW), jnp.float32)]
            + [pltpu.SemaphoreType.DMA for _ in range(4)]
        ),
    )(feat_t, lbl)


RB = 2048


def _onehot_body(feat_ref, lbl_ref, out_ref):
    i = pl.program_id(0)
    rows = feat_ref[...]
    rn = rows * lax.rsqrt(jnp.maximum(
        jnp.sum(rows * rows, axis=1, keepdims=True), 1e-24))
    ext = jnp.concatenate(
        [rn, jnp.ones((RB, 128), jnp.float32)], axis=1).astype(jnp.bfloat16)
    lbl = lbl_ref[...].reshape(RB, 1)
    onehot = (lbl == lax.broadcasted_iota(jnp.int32, (RB, NUM_CLASSES), 1)
              ).astype(jnp.bfloat16)
    part = lax.dot_general(onehot, ext, (((0,), (0,)), ((), ())),
                           preferred_element_type=jnp.float32)

    @pl.when(i == 0)
    def _():
        out_ref[...] = part

    @pl.when(i > 0)
    def _():
        out_ref[...] = out_ref[...] + part


def _onehot_partial(feat, lbl):
    return pl.pallas_call(
        _onehot_body,
        grid=(TC_ROWS // RB,),
        in_specs=[
            pl.BlockSpec((RB, EMB), lambda i: (i, 0)),
            pl.BlockSpec((RB,), lambda i: (i,)),
        ],
        out_specs=pl.BlockSpec((NUM_CLASSES, EMB + 128), lambda i: (0, 0)),
        out_shape=jax.ShapeDtypeStruct((NUM_CLASSES, EMB + 128), jnp.float32),
    )(feat, lbl)


def _proto_body(sums_ref, counts_ref, tcp_ref, w1_ref, b1_ref, w2_ref,
                b2_ref, t_ref):
    sums = (sums_ref[:NUM_CLASSES, :] + sums_ref[NUM_CLASSES:, :]
            + tcp_ref[:, :EMB])
    cnt = (counts_ref[:NUM_CLASSES, 0:1] + counts_ref[NUM_CLASSES:, 0:1]
           + tcp_ref[:, EMB:EMB + 1])
    inv = 1.0 / jnp.maximum(cnt, 1.0)
    protos = sums * inv
    h = jnp.dot(protos, w1_ref[...], preferred_element_type=jnp.float32)
    h = jnp.maximum(h + b1_ref[...], 0.0)
    t = jnp.dot(h, w2_ref[...], preferred_element_type=jnp.float32) + b2_ref[...]
    ss = jnp.sum(t * t, axis=1, keepdims=True)
    t_ref[...] = t * lax.rsqrt(jnp.maximum(ss, 1e-24))


def _proto_mlp(sums, counts, tc_part, W1, b1, W2, b2):
    return pl.pallas_call(
        _proto_body,
        out_shape=jax.ShapeDtypeStruct((NUM_CLASSES, EMB), jnp.float32),
    )(sums, counts, tc_part, W1, b1.reshape(1, EMB), W2, b2.reshape(1, EMB))


BQ = 2048


def _logits_body(q_ref, t_ref, out_ref):
    q = q_ref[...]
    qn = q * lax.rsqrt(jnp.maximum(jnp.sum(q * q, axis=1, keepdims=True), 1e-24))
    out_ref[...] = lax.dot_general(qn, t_ref[...], (((1,), (1,)), ((), ())),
                                   preferred_element_type=jnp.float32)


def _logits(q, t):
    return pl.pallas_call(
        _logits_body,
        grid=(N_QUERY // BQ,),
        in_specs=[
            pl.BlockSpec((BQ, EMB), lambda i: (i, 0)),
            pl.BlockSpec((NUM_CLASSES, EMB), lambda i: (0, 0)),
        ],
        out_specs=pl.BlockSpec((BQ, NUM_CLASSES), lambda i: (i, 0)),
        out_shape=jax.ShapeDtypeStruct((N_QUERY, NUM_CLASSES), jnp.float32),
    )(q, t)


def kernel(support_features, support_labels, query_features, W1, b1, W2, b2):
    lbl = support_labels.astype(jnp.int32)
    feat_t = support_features.reshape(N_SUPPORT // 8, 8, EMB // 128,
                                      128).swapaxes(1, 2)
    sums, counts = _sc_segment(feat_t, lbl)
    tc_part = _onehot_partial(support_features[:TC_ROWS], lbl[:TC_ROWS])
    t = _proto_mlp(sums, counts, tc_part, W1, b1, W2, b2)
    logits = _logits(query_features, t)
    return logits, t


# speedup vs baseline: 1.0964x; 1.0964x over previous
import functools

import jax
import jax.numpy as jnp
from jax import lax
from jax.experimental import pallas as pl
from jax.experimental.pallas import tpu as pltpu
from jax.experimental.pallas import tpu_sc as plsc

NUM_CLASSES = 1024
EMB = 512
N_SUPPORT = 32768
N_QUERY = 16384

NC = 2
NS = 16
L = 16
NW = NC * NS
ROWS_PER_TILE = N_SUPPORT // NW
CHUNKS = EMB // L
GB = 32
NBATCH = ROWS_PER_TILE // GB
NBUF = 4
CLS_STRIPE = NUM_CLASSES // NS
CW = 16


def _sc_segment_body(feat_hbm, lbl_hbm, sums_hbm, counts_hbm,
                     in0, in1, out0, out1,
                     lbi0, lbi1, lbo0, lbo1, ones_v,
                     sums_sh, counts_sh, sg0, sg1, ss0, ss1):
    c = lax.axis_index("c")
    s = lax.axis_index("s")
    tid = c * NS + s
    row_base = tid * ROWS_PER_TILE
    acc_base = c * NUM_CLASSES

    ins = [in0, in1]
    outs = [out0, out1]
    lbis = [lbi0, lbi1]
    lbos = [lbo0, lbo1]
    sgs = [sg0, sg1]
    sss = [ss0, ss1]

    zeros = jnp.zeros((L,), jnp.float32)

    def zero_out0(r, _):
        for k in range(CHUNKS):
            out0[r, pl.ds(k * L, L)] = zeros
        for k in range(CW // L):
            ones_v[r, pl.ds(k * L, L)] = zeros
        return 0
    lax.fori_loop(0, GB, zero_out0, 0)
    stripe = s * CLS_STRIPE
    pltpu.sync_copy(out0, sums_sh.at[pl.ds(stripe, GB)])
    pltpu.sync_copy(out0, sums_sh.at[pl.ds(stripe + GB, GB)])
    pltpu.sync_copy(ones_v, counts_sh.at[pl.ds(stripe, GB)])
    pltpu.sync_copy(ones_v, counts_sh.at[pl.ds(stripe + GB, GB)])

    def fill_ones(r, _):
        for k in range(CW // L):
            ones_v[r, pl.ds(k * L, L)] = jnp.full((L,), 1.0, jnp.float32)
        return 0
    lax.fori_loop(0, GB, fill_ones, 0)

    plsc.subcore_barrier()

    def issue_gather(b, B):
        blk = (row_base + b * GB) // 8
        pltpu.async_copy(feat_hbm.at[pl.ds(blk, GB // 8)], ins[B], sgs[B])
        pltpu.async_copy(lbl_hbm.at[pl.ds(row_base + b * GB, GB)],
                         lbis[B], sgs[B])

    def wait_gather(b, B):
        blk = (row_base + b * GB) // 8
        pltpu.make_async_copy(feat_hbm.at[pl.ds(blk, GB // 8)],
                              ins[B], sgs[B]).wait()
        pltpu.make_async_copy(lbl_hbm.at[pl.ds(row_base + b * GB, GB)],
                              lbis[B], sgs[B]).wait()

    def issue_scatter(B):
        pltpu.async_copy(outs[B], sums_sh.at[lbos[B]], sss[B], add=True)
        pltpu.async_copy(ones_v, counts_sh.at[lbos[B]], sss[B], add=True)

    def wait_scatter(B):
        pltpu.make_async_copy(outs[B], sums_sh.at[lbos[B]], sss[B]).wait()
        pltpu.make_async_copy(ones_v, counts_sh.at[lbos[B]], sss[B]).wait()

    def normalize(BI, BO):
        for g in range(GB // L):
            lbos[BO][pl.ds(g * L, L)] = lbis[BI][pl.ds(g * L, L)]

        def block_body(rb, _):
            for rs in range(8):
                r = rb * 8 + rs
                vs = []
                parts = []
                for k in range(CHUNKS):
                    v = ins[BI][rb, k // 8, rs, pl.ds((k % 8) * L, L)]
                    vs.append(v)
                    parts.append(v * v)
                while len(parts) > 1:
                    rest = [parts[-1]] if len(parts) % 2 else []
                    parts = [parts[i2] + parts[i2 + 1]
                             for i2 in range(0, len(parts) - 1, 2)] + rest
                tv = jnp.broadcast_to(jnp.sum(parts[0]), (L,))
                iv = plsc.bitcast(tv, jnp.int32)
                y = plsc.bitcast(jnp.int32(0x5F3759DF) - (iv >> 1),
                                 jnp.float32)
                for _ in range(3):
                    y = y * (jnp.float32(1.5) - jnp.float32(0.5) * tv * y * y)
                for k in range(CHUNKS):
                    outs[BO][r, pl.ds(k * L, L)] = vs[k] * y
            return 0

        lax.fori_loop(0, GB // 8, block_body, 0)

    issue_gather(0, 0)
    issue_gather(1, 1)

    def pair_body(p, _):
        for sec in range(2):
            b = 2 * p + sec
            wait_gather(b, sec)

            @pl.when(b >= 2)
            def _(sec=sec):
                wait_scatter(sec)
            normalize(sec, sec)
            issue_scatter(sec)

            @pl.when(b + 2 < NBATCH)
            def _(b=b, sec=sec):
                issue_gather(b + 2, sec)
        return 0

    lax.fori_loop(0, NBATCH // 2, pair_body, 0)
    wait_scatter(0)
    wait_scatter(1)

    plsc.subcore_barrier()
    pltpu.sync_copy(sums_sh.at[pl.ds(stripe, CLS_STRIPE)],
                    sums_hbm.at[pl.ds(acc_base + stripe, CLS_STRIPE)])
    pltpu.sync_copy(counts_sh.at[pl.ds(stripe, CLS_STRIPE)],
                    counts_hbm.at[pl.ds(acc_base + stripe, CLS_STRIPE)])


def _sc_segment(feat_t, lbl):
    mesh = plsc.VectorSubcoreMesh(core_axis_name="c", subcore_axis_name="s",
                                  num_cores=NC, num_subcores=NS)
    return pl.kernel(
        _sc_segment_body,
        out_type=(jax.ShapeDtypeStruct((NC * NUM_CLASSES, EMB), jnp.float32),
                  jax.ShapeDtypeStruct((NC * NUM_CLASSES, CW), jnp.float32)),
        mesh=mesh,
        compiler_params=pltpu.CompilerParams(needs_layout_passes=False,
                                             use_tc_tiling_on_sc=False),
        scratch_types=(
            [pltpu.VMEM((GB // 8, EMB // 128, 8, 128), jnp.float32)
             for _ in range(2)]
            + [pltpu.VMEM((GB, EMB), jnp.float32) for _ in range(2)]
            + [pltpu.VMEM((GB,), jnp.int32) for _ in range(4)]
            + [pltpu.VMEM((GB, CW), jnp.float32)]
            + [pltpu.VMEM_SHARED((NUM_CLASSES, EMB), jnp.float32),
               pltpu.VMEM_SHARED((NUM_CLASSES, CW), jnp.float32)]
            + [pltpu.SemaphoreType.DMA for _ in range(4)]
        ),
    )(feat_t, lbl)


BQ = 2048


def _fused_body(sums_ref, counts_ref, w1_ref, b1_ref, w2_ref, b2_ref,
                q_ref, logits_ref, t_ref, t_v):
    i = pl.program_id(0)

    @pl.when(i == 0)
    def _():
        sums = sums_ref[:NUM_CLASSES, :] + sums_ref[NUM_CLASSES:, :]
        cnt = counts_ref[:NUM_CLASSES, :] + counts_ref[NUM_CLASSES:, :]
        inv = 1.0 / jnp.maximum(cnt[:, 0:1], 1.0)
        protos = sums * inv
        h = jnp.dot(protos, w1_ref[...], preferred_element_type=jnp.float32)
        h = jnp.maximum(h + b1_ref[...], 0.0)
        t = jnp.dot(h, w2_ref[...],
                    preferred_element_type=jnp.float32) + b2_ref[...]
        ss = jnp.sum(t * t, axis=1, keepdims=True)
        t = t * lax.rsqrt(jnp.maximum(ss, 1e-24))
        t_v[...] = t
        t_ref[...] = t

    q = q_ref[...]
    qn = q * lax.rsqrt(jnp.maximum(jnp.sum(q * q, axis=1, keepdims=True),
                                   1e-24))
    logits_ref[...] = lax.dot_general(qn, t_v[...], (((1,), (1,)), ((), ())),
                                      preferred_element_type=jnp.float32)


def _proto_and_logits(sums, counts, W1, b1, W2, b2, q):
    return pl.pallas_call(
        _fused_body,
        grid=(N_QUERY // BQ,),
        in_specs=[
            pl.BlockSpec((NC * NUM_CLASSES, EMB), lambda i: (0, 0)),
            pl.BlockSpec((NC * NUM_CLASSES, CW), lambda i: (0, 0)),
            pl.BlockSpec((EMB, EMB), lambda i: (0, 0)),
            pl.BlockSpec((1, EMB), lambda i: (0, 0)),
            pl.BlockSpec((EMB, EMB), lambda i: (0, 0)),
            pl.BlockSpec((1, EMB), lambda i: (0, 0)),
            pl.BlockSpec((BQ, EMB), lambda i: (i, 0)),
        ],
        out_specs=[
            pl.BlockSpec((BQ, NUM_CLASSES), lambda i: (i, 0)),
            pl.BlockSpec((NUM_CLASSES, EMB), lambda i: (0, 0)),
        ],
        out_shape=[
            jax.ShapeDtypeStruct((N_QUERY, NUM_CLASSES), jnp.float32),
            jax.ShapeDtypeStruct((NUM_CLASSES, EMB), jnp.float32),
        ],
        scratch_shapes=[pltpu.VMEM((NUM_CLASSES, EMB), jnp.float32)],
    )(sums, counts, W1, b1.reshape(1, EMB), W2, b2.reshape(1, EMB), q)


def kernel(support_features, support_labels, query_features, W1, b1, W2, b2):
    lbl = support_labels.astype(jnp.int32)
    feat_t = support_features.reshape(N_SUPPORT // 8, 8, EMB // 128,
                                      128).swapaxes(1, 2)
    sums, counts = _sc_segment(feat_t, lbl)
    logits, t = _proto_and_logits(sums, counts, W1, b1, W2, b2,
                                  query_features)
    return logits, t
